# baseline (speedup 1.0000x reference)
import functools

import jax
import jax.numpy as jnp
from jax import lax
from jax.experimental import pallas as pl
from jax.experimental.pallas import tpu as pltpu
from jax.experimental.pallas import tpu_sc as plsc

N = 10000
E = 320000
G = 64
H = 128
HH = H // 2
NP = 10240
CH = 128
NSLAB = 32
CPS = 79
EPAD = NSLAB * CPS * CH
BM = 512
NBLK = NP // BM
STRIPE = NP // 16

_f32 = jnp.float32


NPH = NP // 2
ACC_ROWS = 5136
ZSTRIPE = ACC_ROWS // 16
OSTRIPE = NPH // 16


@functools.cache
def _get_deg_kernel():
    mesh = plsc.VectorSubcoreMesh(core_axis_name="c", subcore_axis_name="s")
    return functools.partial(
        pl.kernel,
        out_type=jax.ShapeDtypeStruct((2, 2, NPH, H), _f32),
        mesh=mesh,
        scratch_types=[
            pltpu.VMEM((CPS, CH), jnp.int32),
            pltpu.VMEM((1, CH), jnp.int32),
            pltpu.VMEM((CH, H), _f32),
            pltpu.VMEM_SHARED((ACC_ROWS, H), _f32),
        ],
    )(_deg_body)


def _deg_body(dst_hbm, ones_hbm, zeros_hbm, out_hbm,
              didx_v, midx_v, ones_v, acc_sh):
    c = lax.axis_index("c")
    s = lax.axis_index("s")
    w = c * 16 + s
    pltpu.sync_copy(ones_hbm, ones_v)
    pltpu.sync_copy(dst_hbm.at[w], didx_v)
    trash = lax.iota(jnp.int32, 16) + NPH

    def make_body(k):
        base = k * NPH

        def body(j, _):
            for i in range(CH // 16):
                d16 = didx_v[j, pl.ds(i * 16, 16)]
                local = d16 - base
                valid = (d16 >= base) & (local < NPH)
                midx_v[0, pl.ds(i * 16, 16)] = jnp.where(valid, local, trash)
            pltpu.sync_copy(ones_v, acc_sh.at[midx_v.at[0]], add=True)
            return 0
        return body

    for k in range(2):
        pltpu.sync_copy(zeros_hbm, acc_sh.at[pl.ds(s * ZSTRIPE, ZSTRIPE)])
        plsc.subcore_barrier()
        lax.fori_loop(0, CPS, make_body(k), 0)
        plsc.subcore_barrier()
        pltpu.sync_copy(acc_sh.at[pl.ds(s * OSTRIPE, OSTRIPE)],
                        out_hbm.at[c, k, pl.ds(s * OSTRIPE, OSTRIPE)])
        plsc.subcore_barrier()


@functools.cache
def _get_conv_kernel():
    mesh = plsc.VectorSubcoreMesh(core_axis_name="c", subcore_axis_name="s")
    return functools.partial(
        pl.kernel,
        out_type=jax.ShapeDtypeStruct((2, 2, NPH, H), _f32),
        mesh=mesh,
        scratch_types=[
            pltpu.VMEM((CPS, CH), jnp.int32),
            pltpu.VMEM((CPS, CH), jnp.int32),
            pltpu.VMEM((3, CH), jnp.int32),
            pltpu.VMEM((3, CH), jnp.int32),
            pltpu.VMEM((3 * CH, H), _f32),
            pltpu.VMEM_SHARED((ACC_ROWS, H), _f32),
            pltpu.SemaphoreType.DMA,
            pltpu.SemaphoreType.DMA,
        ],
    )(_conv_body)


def _conv_body(hn_hbm, he_hbm, src_hbm, dst_hbm, zeros_hbm, out_hbm,
               sidx_v, didx_v, midx_v, msrc_v, rows_v, acc_sh, gsem, ssem):
    c = lax.axis_index("c")
    s = lax.axis_index("s")
    trash = lax.iota(jnp.int32, 16) + NPH
    BLAST = (CPS - 1) % 3

    def remap(j, b, base):
        for i in range(CH // 16):
            d16 = didx_v[j, pl.ds(i * 16, 16)]
            s16 = sidx_v[j, pl.ds(i * 16, 16)]
            local = d16 - base
            valid = (d16 >= base) & (local < NPH)
            midx_v[b, pl.ds(i * 16, 16)] = jnp.where(valid, local, trash)
            msrc_v[b, pl.ds(i * 16, 16)] = jnp.where(valid, s16, 0)

    def gbuf(b):
        return rows_v.at[pl.ds(b * CH, CH)]

    def run_slab(h_hbm, base):
        def body(j, _):
            b = lax.rem(j, 3)
            pltpu.make_async_copy(h_hbm.at[msrc_v.at[b]], gbuf(b),
                                  gsem).wait()

            @pl.when(j >= 1)
            def _():
                bp = lax.rem(j + 2, 3)
                pltpu.make_async_copy(gbuf(bp), acc_sh.at[midx_v.at[bp]],
                                      ssem).wait()
            pltpu.async_copy(gbuf(b), acc_sh.at[midx_v.at[b]], ssem,
                             add=True)

            @pl.when(j + 2 < CPS)
            def _():
                b3 = lax.rem(j + 2, 3)
                remap(j + 2, b3, base)
                pltpu.async_copy(h_hbm.at[msrc_v.at[b3]], gbuf(b3), gsem)
            return 0

        for j in range(2):
            remap(j, j, base)
            pltpu.async_copy(h_hbm.at[msrc_v.at[j]], gbuf(j), gsem)
        lax.fori_loop(0, CPS, body, 0)
        pltpu.make_async_copy(gbuf(BLAST), acc_sh.at[midx_v.at[BLAST]],
                              ssem).wait()

    def sweep(h_hbm, k):
        for slab in range(2):
            pltpu.sync_copy(src_hbm.at[s + slab * 16], sidx_v)
            pltpu.sync_copy(dst_hbm.at[s + slab * 16], didx_v)
            run_slab(h_hbm, k * NPH)

    for k in range(2):
        pltpu.sync_copy(zeros_hbm, acc_sh.at[pl.ds(s * ZSTRIPE, ZSTRIPE)])
        plsc.subcore_barrier()

        @pl.when(c == 0)
        def _(k=k):
            sweep(hn_hbm, k)

        @pl.when(c == 1)
        def _(k=k):
            sweep(he_hbm, k)

        plsc.subcore_barrier()
        pltpu.sync_copy(acc_sh.at[pl.ds(s * OSTRIPE, OSTRIPE)],
                        out_hbm.at[c, k, pl.ds(s * OSTRIPE, OSTRIPE)])
        plsc.subcore_barrier()


def _tc1_body(x_ref, e_ref, deg_ref, wn_ref, we_ref, hn_ref, he_ref, dinv_ref):
    d = deg_ref[...]
    deg = d[0, :, 0:1] + d[1, :, 0:1]
    dinv = lax.rsqrt(deg + 1.0)
    dinv_ref[...] = dinv
    hn_ref[...] = jnp.dot(x_ref[...], wn_ref[...],
                          preferred_element_type=_f32) * dinv
    he_ref[...] = jnp.dot(e_ref[...], we_ref[...],
                          preferred_element_type=_f32) * dinv


def _tc1_call(xp, ep, deg2, Wn1, We1):
    return pl.pallas_call(
        _tc1_body,
        grid=(NBLK,),
        in_specs=[
            pl.BlockSpec((BM, 128), lambda i: (i, 0)),
            pl.BlockSpec((BM, 16), lambda i: (i, 0)),
            pl.BlockSpec((2, BM, H), lambda i: (0, i, 0)),
            pl.BlockSpec((128, H), lambda i: (0, 0)),
            pl.BlockSpec((16, H), lambda i: (0, 0)),
        ],
        out_specs=[
            pl.BlockSpec((BM, H), lambda i: (i, 0)),
            pl.BlockSpec((BM, H), lambda i: (i, 0)),
            pl.BlockSpec((BM, 1), lambda i: (i, 0)),
        ],
        out_shape=[
            jax.ShapeDtypeStruct((NP, H), _f32),
            jax.ShapeDtypeStruct((NP, H), _f32),
            jax.ShapeDtypeStruct((NP, 1), _f32),
        ],
    )(xp, ep, deg2, Wn1, We1)


def _tc2_body(acc_ref, hn1_ref, he1_ref, dinv_ref,
              wn_ref, we_ref, bn1_ref, be1_ref, hn2_ref, he2_ref):
    a = acc_ref[...]
    dinv = dinv_ref[...]
    n1 = jax.nn.relu(dinv * (a[0] + hn1_ref[...]) + bn1_ref[...])
    e1 = jax.nn.relu(dinv * (a[1] + he1_ref[...]) + be1_ref[...])
    hn2_ref[...] = jnp.dot(n1, wn_ref[...], preferred_element_type=_f32) * dinv
    he2_ref[...] = jnp.dot(e1, we_ref[...], preferred_element_type=_f32) * dinv


def _tc2_call(acc1, hn1, he1, dinv, Wn2, We2, bn1, be1):
    return pl.pallas_call(
        _tc2_body,
        grid=(NBLK,),
        in_specs=[
            pl.BlockSpec((2, BM, H), lambda i: (0, i, 0)),
            pl.BlockSpec((BM, H), lambda i: (i, 0)),
            pl.BlockSpec((BM, H), lambda i: (i, 0)),
            pl.BlockSpec((BM, 1), lambda i: (i, 0)),
            pl.BlockSpec((H, H), lambda i: (0, 0)),
            pl.BlockSpec((H, H), lambda i: (0, 0)),
            pl.BlockSpec((1, H), lambda i: (0, 0)),
            pl.BlockSpec((1, H), lambda i: (0, 0)),
        ],
        out_specs=[
            pl.BlockSpec((BM, H), lambda i: (i, 0)),
            pl.BlockSpec((BM, H), lambda i: (i, 0)),
        ],
        out_shape=[
            jax.ShapeDtypeStruct((NP, H), _f32),
            jax.ShapeDtypeStruct((NP, H), _f32),
        ],
    )(acc1, hn1, he1, dinv, Wn2, We2, bn1, be1)


def _tc3_body(acc_ref, hn2_ref, he2_ref, dinv_ref,
              bn2_ref, be2_ref, nb_ref, eb_ref, w_ref,
              ww1_ref, bw1_ref, ww2_ref, bw2_ref,
              wr1_ref, br1_ref, wr2_ref, br2_ref, out_ref,
              pn_acc, pe_acc, cn_acc, ce_acc):
    i = pl.program_id(0)

    @pl.when(i == 0)
    def _():
        pn_acc[...] = jnp.zeros((G, H), _f32)
        pe_acc[...] = jnp.zeros((G, H), _f32)
        cn_acc[...] = jnp.zeros((G, H), _f32)
        ce_acc[...] = jnp.zeros((G, H), _f32)

    a = acc_ref[...]
    dinv = dinv_ref[...]
    nh = dinv * (a[0] + hn2_ref[...]) + bn2_ref[...]
    eh = dinv * (a[1] + he2_ref[...]) + be2_ref[...]
    gids = lax.broadcasted_iota(jnp.int32, (G, BM), 0)
    ones = jnp.ones((BM, H), _f32)
    ohn = (gids == jnp.broadcast_to(nb_ref[...], (G, BM))).astype(_f32)
    ohe = (gids == jnp.broadcast_to(eb_ref[...], (G, BM))).astype(_f32)
    pn_acc[...] += jnp.dot(ohn, nh, preferred_element_type=_f32)
    pe_acc[...] += jnp.dot(ohe, eh, preferred_element_type=_f32)
    cn_acc[...] += jnp.dot(ohn, ones, preferred_element_type=_f32)
    ce_acc[...] += jnp.dot(ohe, ones, preferred_element_type=_f32)

    @pl.when(i == NBLK - 1)
    def _():
        pn = pn_acc[...] / jnp.maximum(cn_acc[...], 1.0)
        pe = pe_acc[...] / jnp.maximum(ce_acc[...], 1.0)
        wh = jax.nn.relu(jnp.dot(w_ref[...], ww1_ref[...],
                                 preferred_element_type=_f32) + bw1_ref[...])
        wh = jnp.dot(wh, ww2_ref[...], preferred_element_type=_f32) + bw2_ref[...]
        comb = jnp.concatenate([pn, pe, wh], axis=1)
        oh = jax.nn.relu(jnp.dot(comb, wr1_ref[...],
                                 preferred_element_type=_f32) + br1_ref[...])
        out_ref[...] = jnp.dot(oh, wr2_ref[...],
                               preferred_element_type=_f32) + br2_ref[...]


def _tc3_call(acc2, hn2, he2, dinv, bn2, be2, nb, eb, weather,
              Ww1, bw1, Ww2, bw2, Wr1, br1, Wr2, br2):
    def full(shape):
        nzero = len(shape)
        return pl.BlockSpec(shape, lambda i, _n=nzero: (0,) * _n)
    return pl.pallas_call(
        _tc3_body,
        grid=(NBLK,),
        in_specs=[
            pl.BlockSpec((2, BM, H), lambda i: (0, i, 0)),
            pl.BlockSpec((BM, H), lambda i: (i, 0)),
            pl.BlockSpec((BM, H), lambda i: (i, 0)),
            pl.BlockSpec((BM, 1), lambda i: (i, 0)),
            full((1, H)), full((1, H)),
            pl.BlockSpec((1, BM), lambda i: (0, i)),
            pl.BlockSpec((1, BM), lambda i: (0, i)),
            full((G, 16)),
            full((16, H)), full((1, H)), full((H, H)), full((1, H)),
            full((3 * H, H)), full((1, H)), full((H, 1)), full((1, 1)),
        ],
        out_specs=pl.BlockSpec((G, 1), lambda i: (0, 0)),
        out_shape=jax.ShapeDtypeStruct((G, 1), _f32),
        scratch_shapes=[pltpu.VMEM((G, H), _f32) for _ in range(4)],
    )(acc2, hn2, he2, dinv, bn2, be2, nb, eb,
      weather, Ww1, bw1, Ww2, bw2, Wr1, br1, Wr2, br2)


def kernel(node_features, edge_attr, weather_attr, edge_index, node_batch,
           edge_batch, Wn1, bn1, Wn2, bn2, We1, be1, We2, be2, Ww1, bw1,
           Ww2, bw2, Wr1, br1, Wr2, br2):
    xp = jnp.pad(node_features, ((0, NP - N), (0, 0)))
    ep = jnp.pad(edge_attr, ((0, NP - N), (0, 0)))
    nb = jnp.pad(node_batch.astype(jnp.int32), (0, NP - N),
                 constant_values=G).reshape(1, NP)
    eb = jnp.pad(edge_batch.astype(jnp.int32), (0, NP - N),
                 constant_values=G).reshape(1, NP)
    src = jnp.pad(edge_index[0].astype(jnp.int32), (0, EPAD - E),
                  constant_values=0).reshape(NSLAB, CPS, CH)
    dst = jnp.pad(edge_index[1].astype(jnp.int32), (0, EPAD - E),
                  constant_values=NP - 1).reshape(NSLAB, CPS, CH)
    ones_rows = jnp.ones((CH, H), _f32)
    zeros_rows = jnp.zeros((ZSTRIPE, H), _f32)

    deg2 = _get_deg_kernel()(dst, ones_rows, zeros_rows).reshape(2, NP, H)
    hn1, he1, dinv = _tc1_call(xp, ep, deg2, Wn1, We1)
    acc1 = _get_conv_kernel()(hn1, he1, src, dst, zeros_rows).reshape(2, NP, H)
    hn2, he2 = _tc2_call(acc1, hn1, he1, dinv, Wn2, We2, bn1.reshape(1, H),
                         be1.reshape(1, H))
    acc2 = _get_conv_kernel()(hn2, he2, src, dst, zeros_rows).reshape(2, NP, H)
    out = _tc3_call(acc2, hn2, he2, dinv, bn2.reshape(1, H),
                    be2.reshape(1, H), nb, eb, weather_attr,
                    Ww1, bw1.reshape(1, H), Ww2, bw2.reshape(1, H),
                    Wr1, br1.reshape(1, H), Wr2, br2.reshape(1, 1))
    return out


# speedup vs baseline: 15.8373x; 15.8373x over previous
import functools

import jax
import jax.numpy as jnp
from jax import lax
from jax.experimental import pallas as pl
from jax.experimental.pallas import tpu as pltpu
from jax.experimental.pallas import tpu_sc as plsc

N = 10000
E = 320000
G = 64
H = 128
HH = H // 2
NP = 10240
CH = 128
NSLAB = 32
CPS = 79
EPAD = NSLAB * CPS * CH
BM = 512
NBLK = NP // BM
STRIPE = NP // 16

_f32 = jnp.float32


NPH = NP // 2
ACC_ROWS = 5136
ZSTRIPE = ACC_ROWS // 16
OSTRIPE = NPH // 16


@functools.cache
def _get_deg_kernel():
    mesh = plsc.VectorSubcoreMesh(core_axis_name="c", subcore_axis_name="s")
    return functools.partial(
        pl.kernel,
        out_type=jax.ShapeDtypeStruct((2, 2, NPH, H), _f32),
        mesh=mesh,
        scratch_types=[
            pltpu.VMEM((CPS, CH), jnp.int32),
            pltpu.VMEM((1, CH), jnp.int32),
            pltpu.VMEM((CH, H), _f32),
            pltpu.VMEM_SHARED((ACC_ROWS, H), _f32),
        ],
    )(_deg_body)


def _deg_body(dst_hbm, ones_hbm, zeros_hbm, out_hbm,
              didx_v, midx_v, ones_v, acc_sh):
    c = lax.axis_index("c")
    s = lax.axis_index("s")
    w = c * 16 + s
    pltpu.sync_copy(ones_hbm, ones_v)
    pltpu.sync_copy(dst_hbm.at[w], didx_v)
    trash = lax.iota(jnp.int32, 16) + NPH

    def make_body(k):
        base = k * NPH

        def body(j, _):
            for i in range(CH // 16):
                d16 = didx_v[j, pl.ds(i * 16, 16)]
                local = d16 - base
                valid = (d16 >= base) & (local < NPH)
                midx_v[0, pl.ds(i * 16, 16)] = jnp.where(valid, local, trash)
            pltpu.sync_copy(ones_v, acc_sh.at[midx_v.at[0]], add=True)
            return 0
        return body

    for k in range(2):
        pltpu.sync_copy(zeros_hbm, acc_sh.at[pl.ds(s * ZSTRIPE, ZSTRIPE)])
        plsc.subcore_barrier()
        lax.fori_loop(0, CPS, make_body(k), 0)
        plsc.subcore_barrier()
        pltpu.sync_copy(acc_sh.at[pl.ds(s * OSTRIPE, OSTRIPE)],
                        out_hbm.at[c, k, pl.ds(s * OSTRIPE, OSTRIPE)])
        plsc.subcore_barrier()


@functools.cache
def _get_conv_kernel():
    mesh = plsc.VectorSubcoreMesh(core_axis_name="c", subcore_axis_name="s")
    return functools.partial(
        pl.kernel,
        out_type=jax.ShapeDtypeStruct((2, 2, NPH, H), _f32),
        mesh=mesh,
        scratch_types=[
            pltpu.VMEM((CPS, CH), jnp.int32),
            pltpu.VMEM((CPS, CH), jnp.int32),
            pltpu.VMEM((3, CH), jnp.int32),
            pltpu.VMEM((3 * CH, H), _f32),
            pltpu.VMEM_SHARED((ACC_ROWS, H), _f32),
            pltpu.SemaphoreType.DMA,
            pltpu.SemaphoreType.DMA,
        ],
    )(_conv_body)


def _conv_body(hn_hbm, he_hbm, src_hbm, dst_hbm, zeros_hbm, out_hbm,
               sidx_v, didx_v, midx_v, rows_v, acc_sh, gsem, ssem):
    c = lax.axis_index("c")
    s = lax.axis_index("s")
    trash = lax.iota(jnp.int32, 16) + NPH
    BLAST = (CPS - 1) % 3

    def remap(j, b, base):
        for i in range(CH // 16):
            d16 = didx_v[j, pl.ds(i * 16, 16)]
            local = d16 - base
            valid = (d16 >= base) & (local < NPH)
            midx_v[b, pl.ds(i * 16, 16)] = jnp.where(valid, local, trash)

    def gbuf(b):
        return rows_v.at[pl.ds(b * CH, CH)]

    def run_slab(h_hbm, base):
        def body(j, _):
            b = lax.rem(j, 3)
            pltpu.make_async_copy(h_hbm.at[sidx_v.at[j]], gbuf(b),
                                  gsem).wait()

            @pl.when(j >= 1)
            def _():
                bp = lax.rem(j + 2, 3)
                pltpu.make_async_copy(gbuf(bp), acc_sh.at[midx_v.at[bp]],
                                      ssem).wait()
            pltpu.async_copy(gbuf(b), acc_sh.at[midx_v.at[b]], ssem,
                             add=True)

            @pl.when(j + 2 < CPS)
            def _():
                b3 = lax.rem(j + 2, 3)
                remap(j + 2, b3, base)
                pltpu.async_copy(h_hbm.at[sidx_v.at[j + 2]], gbuf(b3), gsem)
            return 0

        for j in range(2):
            remap(j, j, base)
            pltpu.async_copy(h_hbm.at[sidx_v.at[j]], gbuf(j), gsem)
        lax.fori_loop(0, CPS, body, 0)
        pltpu.make_async_copy(gbuf(BLAST), acc_sh.at[midx_v.at[BLAST]],
                              ssem).wait()

    def sweep(h_hbm, k):
        for slab in range(2):
            pltpu.sync_copy(src_hbm.at[s + slab * 16], sidx_v)
            pltpu.sync_copy(dst_hbm.at[s + slab * 16], didx_v)
            run_slab(h_hbm, k * NPH)

    for k in range(2):
        pltpu.sync_copy(zeros_hbm, acc_sh.at[pl.ds(s * ZSTRIPE, ZSTRIPE)])
        plsc.subcore_barrier()

        @pl.when(c == 0)
        def _(k=k):
            sweep(hn_hbm, k)

        @pl.when(c == 1)
        def _(k=k):
            sweep(he_hbm, k)

        plsc.subcore_barrier()
        pltpu.sync_copy(acc_sh.at[pl.ds(s * OSTRIPE, OSTRIPE)],
                        out_hbm.at[c, k, pl.ds(s * OSTRIPE, OSTRIPE)])
        plsc.subcore_barrier()


def _tc1_body(x_ref, e_ref, deg_ref, wn_ref, we_ref, hn_ref, he_ref, dinv_ref):
    d = deg_ref[...]
    deg = d[0, :, 0:1] + d[1, :, 0:1]
    dinv = lax.rsqrt(deg + 1.0)
    dinv_ref[...] = dinv
    hn_ref[...] = jnp.dot(x_ref[...], wn_ref[...],
                          preferred_element_type=_f32) * dinv
    he_ref[...] = jnp.dot(e_ref[...], we_ref[...],
                          preferred_element_type=_f32) * dinv


def _tc1_call(xp, ep, deg2, Wn1, We1):
    return pl.pallas_call(
        _tc1_body,
        grid=(NBLK,),
        in_specs=[
            pl.BlockSpec((BM, 128), lambda i: (i, 0)),
            pl.BlockSpec((BM, 16), lambda i: (i, 0)),
            pl.BlockSpec((2, BM, H), lambda i: (0, i, 0)),
            pl.BlockSpec((128, H), lambda i: (0, 0)),
            pl.BlockSpec((16, H), lambda i: (0, 0)),
        ],
        out_specs=[
            pl.BlockSpec((BM, H), lambda i: (i, 0)),
            pl.BlockSpec((BM, H), lambda i: (i, 0)),
            pl.BlockSpec((BM, 1), lambda i: (i, 0)),
        ],
        out_shape=[
            jax.ShapeDtypeStruct((NP, H), _f32),
            jax.ShapeDtypeStruct((NP, H), _f32),
            jax.ShapeDtypeStruct((NP, 1), _f32),
        ],
    )(xp, ep, deg2, Wn1, We1)


def _tc2_body(acc_ref, hn1_ref, he1_ref, dinv_ref,
              wn_ref, we_ref, bn1_ref, be1_ref, hn2_ref, he2_ref):
    a = acc_ref[...]
    dinv = dinv_ref[...]
    n1 = jax.nn.relu(dinv * (a[0] + hn1_ref[...]) + bn1_ref[...])
    e1 = jax.nn.relu(dinv * (a[1] + he1_ref[...]) + be1_ref[...])
    hn2_ref[...] = jnp.dot(n1, wn_ref[...], preferred_element_type=_f32) * dinv
    he2_ref[...] = jnp.dot(e1, we_ref[...], preferred_element_type=_f32) * dinv


def _tc2_call(acc1, hn1, he1, dinv, Wn2, We2, bn1, be1):
    return pl.pallas_call(
        _tc2_body,
        grid=(NBLK,),
        in_specs=[
            pl.BlockSpec((2, BM, H), lambda i: (0, i, 0)),
            pl.BlockSpec((BM, H), lambda i: (i, 0)),
            pl.BlockSpec((BM, H), lambda i: (i, 0)),
            pl.BlockSpec((BM, 1), lambda i: (i, 0)),
            pl.BlockSpec((H, H), lambda i: (0, 0)),
            pl.BlockSpec((H, H), lambda i: (0, 0)),
            pl.BlockSpec((1, H), lambda i: (0, 0)),
            pl.BlockSpec((1, H), lambda i: (0, 0)),
        ],
        out_specs=[
            pl.BlockSpec((BM, H), lambda i: (i, 0)),
            pl.BlockSpec((BM, H), lambda i: (i, 0)),
        ],
        out_shape=[
            jax.ShapeDtypeStruct((NP, H), _f32),
            jax.ShapeDtypeStruct((NP, H), _f32),
        ],
    )(acc1, hn1, he1, dinv, Wn2, We2, bn1, be1)


def _tc3_body(acc_ref, hn2_ref, he2_ref, dinv_ref,
              bn2_ref, be2_ref, nb_ref, eb_ref, w_ref,
              ww1_ref, bw1_ref, ww2_ref, bw2_ref,
              wr1_ref, br1_ref, wr2_ref, br2_ref, out_ref,
              pn_acc, pe_acc, cn_acc, ce_acc):
    i = pl.program_id(0)

    @pl.when(i == 0)
    def _():
        pn_acc[...] = jnp.zeros((G, H), _f32)
        pe_acc[...] = jnp.zeros((G, H), _f32)
        cn_acc[...] = jnp.zeros((G, H), _f32)
        ce_acc[...] = jnp.zeros((G, H), _f32)

    a = acc_ref[...]
    dinv = dinv_ref[...]
    nh = dinv * (a[0] + hn2_ref[...]) + bn2_ref[...]
    eh = dinv * (a[1] + he2_ref[...]) + be2_ref[...]
    gids = lax.broadcasted_iota(jnp.int32, (G, BM), 0)
    ones = jnp.ones((BM, H), _f32)
    ohn = (gids == jnp.broadcast_to(nb_ref[...], (G, BM))).astype(_f32)
    ohe = (gids == jnp.broadcast_to(eb_ref[...], (G, BM))).astype(_f32)
    pn_acc[...] += jnp.dot(ohn, nh, preferred_element_type=_f32)
    pe_acc[...] += jnp.dot(ohe, eh, preferred_element_type=_f32)
    cn_acc[...] += jnp.dot(ohn, ones, preferred_element_type=_f32)
    ce_acc[...] += jnp.dot(ohe, ones, preferred_element_type=_f32)

    @pl.when(i == NBLK - 1)
    def _():
        pn = pn_acc[...] / jnp.maximum(cn_acc[...], 1.0)
        pe = pe_acc[...] / jnp.maximum(ce_acc[...], 1.0)
        wh = jax.nn.relu(jnp.dot(w_ref[...], ww1_ref[...],
                                 preferred_element_type=_f32) + bw1_ref[...])
        wh = jnp.dot(wh, ww2_ref[...], preferred_element_type=_f32) + bw2_ref[...]
        comb = jnp.concatenate([pn, pe, wh], axis=1)
        oh = jax.nn.relu(jnp.dot(comb, wr1_ref[...],
                                 preferred_element_type=_f32) + br1_ref[...])
        out_ref[...] = jnp.dot(oh, wr2_ref[...],
                               preferred_element_type=_f32) + br2_ref[...]


def _tc3_call(acc2, hn2, he2, dinv, bn2, be2, nb, eb, weather,
              Ww1, bw1, Ww2, bw2, Wr1, br1, Wr2, br2):
    def full(shape):
        nzero = len(shape)
        return pl.BlockSpec(shape, lambda i, _n=nzero: (0,) * _n)
    return pl.pallas_call(
        _tc3_body,
        grid=(NBLK,),
        in_specs=[
            pl.BlockSpec((2, BM, H), lambda i: (0, i, 0)),
            pl.BlockSpec((BM, H), lambda i: (i, 0)),
            pl.BlockSpec((BM, H), lambda i: (i, 0)),
            pl.BlockSpec((BM, 1), lambda i: (i, 0)),
            full((1, H)), full((1, H)),
            pl.BlockSpec((1, BM), lambda i: (0, i)),
            pl.BlockSpec((1, BM), lambda i: (0, i)),
            full((G, 16)),
            full((16, H)), full((1, H)), full((H, H)), full((1, H)),
            full((3 * H, H)), full((1, H)), full((H, 1)), full((1, 1)),
        ],
        out_specs=pl.BlockSpec((G, 1), lambda i: (0, 0)),
        out_shape=jax.ShapeDtypeStruct((G, 1), _f32),
        scratch_shapes=[pltpu.VMEM((G, H), _f32) for _ in range(4)],
    )(acc2, hn2, he2, dinv, bn2, be2, nb, eb,
      weather, Ww1, bw1, Ww2, bw2, Wr1, br1, Wr2, br2)


def kernel(node_features, edge_attr, weather_attr, edge_index, node_batch,
           edge_batch, Wn1, bn1, Wn2, bn2, We1, be1, We2, be2, Ww1, bw1,
           Ww2, bw2, Wr1, br1, Wr2, br2):
    xp = jnp.pad(node_features, ((0, NP - N), (0, 0)))
    ep = jnp.pad(edge_attr, ((0, NP - N), (0, 0)))
    nb = jnp.pad(node_batch.astype(jnp.int32), (0, NP - N),
                 constant_values=G).reshape(1, NP)
    eb = jnp.pad(edge_batch.astype(jnp.int32), (0, NP - N),
                 constant_values=G).reshape(1, NP)
    src = jnp.pad(edge_index[0].astype(jnp.int32), (0, EPAD - E),
                  constant_values=0).reshape(NSLAB, CPS, CH)
    dst = jnp.pad(edge_index[1].astype(jnp.int32), (0, EPAD - E),
                  constant_values=NP - 1).reshape(NSLAB, CPS, CH)
    ones_rows = jnp.ones((CH, H), _f32)
    zeros_rows = jnp.zeros((ZSTRIPE, H), _f32)

    deg2 = _get_deg_kernel()(dst, ones_rows, zeros_rows).reshape(2, NP, H)
    hn1, he1, dinv = _tc1_call(xp, ep, deg2, Wn1, We1)
    acc1 = _get_conv_kernel()(hn1, he1, src, dst, zeros_rows).reshape(2, NP, H)
    hn2, he2 = _tc2_call(acc1, hn1, he1, dinv, Wn2, We2, bn1.reshape(1, H),
                         be1.reshape(1, H))
    acc2 = _get_conv_kernel()(hn2, he2, src, dst, zeros_rows).reshape(2, NP, H)
    out = _tc3_call(acc2, hn2, he2, dinv, bn2.reshape(1, H),
                    be2.reshape(1, H), nb, eb, weather_attr,
                    Ww1, bw1.reshape(1, H), Ww2, bw2.reshape(1, H),
                    Wr1, br1.reshape(1, H), Wr2, br2.reshape(1, 1))
    return out
